# Initial kernel scaffold; baseline (speedup 1.0000x reference)
import jax
import jax.numpy as jnp
from jax.experimental import pallas as pl


def kernel(x, scores, score_embeddings):
    raise NotImplementedError("write your pallas kernel here")


# speedup vs baseline: 2.4934x; 2.4934x over previous
import functools

import jax
import jax.numpy as jnp
from jax.experimental import pallas as pl
from jax.experimental.pallas import tpu as pltpu

_ROWS = 16384
_D = 2048
_BLK = 1024
_NBLK = _ROWS // _BLK
_TBL_PAD = 16


def _body(scores_ref, x_ref, tbl_ref, o_ref):
    s = scores_ref[0, 0, :]
    onehot = (s[:, None] == jax.lax.broadcasted_iota(jnp.int32, (1, _TBL_PAD), 1)
              ).astype(jnp.float32)
    emb = jnp.dot(onehot, tbl_ref[...], preferred_element_type=jnp.float32)
    o_ref[...] = x_ref[...] + emb


@jax.jit
def _run(x2d, scores3d, tbl_pad):
    return pl.pallas_call(
        _body,
        grid=(_NBLK,),
        in_specs=[
            pl.BlockSpec((1, 1, _BLK), lambda i: (i, 0, 0)),
            pl.BlockSpec((_BLK, _D), lambda i: (i, 0)),
            pl.BlockSpec((_TBL_PAD, _D), lambda i: (0, 0)),
        ],
        out_specs=pl.BlockSpec((_BLK, _D), lambda i: (i, 0)),
        out_shape=jax.ShapeDtypeStruct((_ROWS, _D), jnp.float32),
        compiler_params=pltpu.CompilerParams(
            dimension_semantics=("arbitrary",),
        ),
    )(scores3d, x2d, tbl_pad)


def kernel(x, scores, score_embeddings):
    b, n, d = x.shape
    x2d = x.reshape(b * n, d)
    scores3d = scores.reshape(_NBLK, 1, _BLK)
    tbl_pad = jnp.pad(score_embeddings,
                      ((0, _TBL_PAD - score_embeddings.shape[0]), (0, 0)))
    out = _run(x2d, scores3d, tbl_pad)
    return out.reshape(b, n, d)
